# Initial kernel scaffold; baseline (speedup 1.0000x reference)
import jax
import jax.numpy as jnp
from jax.experimental import pallas as pl


def kernel(img_ids, img_loc, input_ids, token_type_ids, word_emb, pos_emb, tok_emb, img_W, img_b, loc_W, loc_b, ln_feat_g, ln_feat_b, ln_loc_g, ln_loc_b, ln_img_g, ln_img_b, ln_g, ln_b):
    raise NotImplementedError("write your pallas kernel here")


# speedup vs baseline: 4.1170x; 4.1170x over previous
import functools

import jax
import jax.numpy as jnp
from jax.experimental import pallas as pl
from jax.experimental.pallas import tpu as pltpu

B = 1024
HIDDEN = 1024
VFEAT = 2048
MAX_REGION = 36
MAX_SEQ = 52
NUM_POS = 54
NV = MAX_REGION + 1
NCOL = NV + MAX_SEQ

BB = 16


def _ln(x, g, b, eps=1e-12):
    m = jnp.mean(x, axis=-1, keepdims=True)
    xc = x - m
    v = jnp.mean(xc * xc, axis=-1, keepdims=True)
    return xc * jax.lax.rsqrt(v + eps) * g + b


def _fused_kernel(img_ref, loc_ref, ids_ref, tt_ref,
                  word_ref, pos_ref, tok_ref,
                  imgW_ref, imgb_ref, locW_ref, locb_ref,
                  lnf_g_ref, lnf_b_ref, lnl_g_ref, lnl_b_ref,
                  lni_g_ref, lni_b_ref, ln_g_ref, ln_b_ref,
                  out_ref):
    x = img_ref[:, 1:, :].reshape(BB * MAX_REGION, VFEAT).astype(jnp.bfloat16)
    y = jax.lax.dot_general(
        x, imgW_ref[:],
        dimension_numbers=(((1,), (0,)), ((), ())),
        preferred_element_type=jnp.float32,
    ) + imgb_ref[:]
    a = _ln(y, lnf_g_ref[:], lnf_b_ref[:])

    xl = loc_ref[:, 1:, :].reshape(BB * MAX_REGION, 5)
    yl = jax.lax.dot_general(
        xl, locW_ref[:],
        dimension_numbers=(((1,), (0,)), ((), ())),
        preferred_element_type=jnp.float32,
    ) + locb_ref[:]
    al = _ln(yl, lnl_g_ref[:], lnl_b_ref[:])

    v = _ln(a + al, lni_g_ref[:], lni_b_ref[:])
    c_vis = word_ref[49:50, :] + pos_ref[1:2, :] + tok_ref[0:1, :]
    out_vis = _ln(v + c_vis, ln_g_ref[:], ln_b_ref[:])
    out_ref[:, 1:NV, :] = out_vis.reshape(BB, MAX_REGION, HIDDEN)

    r0 = word_ref[47:48, :] + pos_ref[0:1, :] + tok_ref[0:1, :]
    r0 = _ln(r0, ln_g_ref[:], ln_b_ref[:])
    out_ref[:, 0:1, :] = jnp.broadcast_to(r0[None, :, :], (BB, 1, HIDDEN))

    n2 = BB * MAX_SEQ
    ids_f = ids_ref[:]
    tt_f = tt_ref[:] + 1
    oh_w = (jax.lax.broadcasted_iota(jnp.int32, (n2, 50), 1) == ids_f
            ).astype(jnp.float32)
    oh_t = (jax.lax.broadcasted_iota(jnp.int32, (n2, 4), 1) == tt_f
            ).astype(jnp.float32)
    words = jax.lax.dot_general(
        oh_w, word_ref[:], dimension_numbers=(((1,), (0,)), ((), ())),
        preferred_element_type=jnp.float32)
    toks = jax.lax.dot_general(
        oh_t, tok_ref[:], dimension_numbers=(((1,), (0,)), ((), ())),
        preferred_element_type=jnp.float32)
    s = (words + toks).reshape(BB, MAX_SEQ, HIDDEN) + pos_ref[2:NUM_POS, :][None]
    out_ref[:, NV:, :] = _ln(s, ln_g_ref[:], ln_b_ref[:])


def kernel(img_ids, img_loc, input_ids, token_type_ids, word_emb, pos_emb,
           tok_emb, img_W, img_b, loc_W, loc_b, ln_feat_g, ln_feat_b,
           ln_loc_g, ln_loc_b, ln_img_g, ln_img_b, ln_g, ln_b):
    imgW_t = img_W.T.astype(jnp.bfloat16)
    locW_t = loc_W.T
    row = lambda p: p.reshape(1, HIDDEN)
    ids_flat = input_ids.at[:, 0].set(48).reshape(B * MAX_SEQ, 1)
    tt_flat = token_type_ids.reshape(B * MAX_SEQ, 1)

    grid = (B // BB,)
    resident = lambda shape: pl.BlockSpec(shape, lambda i: (0,) * len(shape))
    out = pl.pallas_call(
        _fused_kernel,
        grid=grid,
        in_specs=[
            pl.BlockSpec((BB, NV, VFEAT), lambda i: (i, 0, 0)),
            pl.BlockSpec((BB, NV, 5), lambda i: (i, 0, 0)),
            pl.BlockSpec((BB * MAX_SEQ, 1), lambda i: (i, 0)),
            pl.BlockSpec((BB * MAX_SEQ, 1), lambda i: (i, 0)),
            resident((50, HIDDEN)),
            resident((NUM_POS, HIDDEN)),
            resident((4, HIDDEN)),
            resident((VFEAT, HIDDEN)),
            resident((1, HIDDEN)),
            resident((5, HIDDEN)),
            resident((1, HIDDEN)),
            resident((1, HIDDEN)),
            resident((1, HIDDEN)),
            resident((1, HIDDEN)),
            resident((1, HIDDEN)),
            resident((1, HIDDEN)),
            resident((1, HIDDEN)),
            resident((1, HIDDEN)),
            resident((1, HIDDEN)),
        ],
        out_specs=pl.BlockSpec((BB, NCOL, HIDDEN), lambda i: (i, 0, 0)),
        out_shape=jax.ShapeDtypeStruct((B, NCOL, HIDDEN), jnp.float32),
        compiler_params=pltpu.CompilerParams(
            dimension_semantics=("arbitrary",),
        ),
    )(img_ids, img_loc, ids_flat, tt_flat, word_emb, pos_emb,
      tok_emb, imgW_t, row(img_b), locW_t, row(loc_b), row(ln_feat_g),
      row(ln_feat_b), row(ln_loc_g), row(ln_loc_b), row(ln_img_g),
      row(ln_img_b), row(ln_g), row(ln_b))
    return out
